# Initial kernel scaffold; baseline (speedup 1.0000x reference)
import jax
import jax.numpy as jnp
from jax.experimental import pallas as pl


def kernel(x, edge_index, edge_type, W1, root1, b1, g1, beta1, W2, root2, b2, g2, beta2):
    raise NotImplementedError("write your pallas kernel here")


# speedup vs baseline: 11.1550x; 11.1550x over previous
import functools

import jax
import jax.numpy as jnp
from jax import lax
from jax.experimental import pallas as pl
from jax.experimental.pallas import tpu as pltpu
from jax.experimental.pallas import tpu_sc as plsc

N = 10000
E = 320000
D = 128
R = 8
EPS = 1e-5

NC = 2
NS = 16
K_PER_SC = 5
CHUNK = 1024
ROWS = CHUNK * R
ROWSP = ROWS + 16
SROWS = NC * K_PER_SC * ROWS
EPT = E // NS
EBLK = 2000
NEB = EPT // EBLK
B = 128
RING = 32
RPT = ROWS // NS


def _sc_body(x_hbm, src_hbm, dst_hbm, typ_hbm, s_hbm, acc_sh, dstv, typv,
             srcv, sel_src, sel_row, rows_v):
    cid = lax.axis_index("c")
    sid = lax.axis_index("s")
    iota16 = lax.iota(jnp.int32, 16)
    zero16 = jnp.zeros((16,), jnp.float32)
    one16 = jnp.ones((16,), jnp.float32)
    RMASK = RING * B - 1

    def zero_rows_v():
        def zr(j, carry):
            for cc in range(D // 16):
                rows_v[j, pl.ds(cc * 16, 16)] = zero16
            return carry
        lax.fori_loop(0, 64, zr, 0)

    ebase = sid * EPT
    db = sid * RPT

    def flush(k0, k1):
        def batch_body(k, carry):
            kr = k & (RING - 1)
            pltpu.sync_copy(x_hbm.at[sel_src.at[kr]], rows_v)
            pltpu.sync_copy(rows_v, acc_sh.at[sel_row.at[kr]], add=True)
            return carry
        lax.fori_loop(k0, k1, batch_body, 0)
        return k1

    def chunk_body(ci, carry):
        c = cid * K_PER_SC + ci
        lo = c * CHUNK

        zero_rows_v()
        for t in range(RPT // 64):
            pltpu.sync_copy(rows_v.at[pl.ds(0, 64)],
                            acc_sh.at[pl.ds(db + t * 64, 64)])
        plsc.subcore_barrier()

        def block_body(bi, carry):
            cnt, kdone = carry
            eoff = ebase + bi * EBLK
            pltpu.sync_copy(dst_hbm.at[pl.ds(eoff, EBLK)], dstv)
            pltpu.sync_copy(typ_hbm.at[pl.ds(eoff, EBLK)], typv)
            pltpu.sync_copy(src_hbm.at[pl.ds(eoff, EBLK)], srcv)

            def scan_body(i, cnt):
                d = dstv[pl.ds(i * 16, 16)]
                t = typv[pl.ds(i * 16, 16)]
                s = srcv[pl.ds(i * 16, 16)]
                dl = d - lo
                m = (dl >= 0) & (dl < CHUNK)
                mi = m.astype(jnp.int32)
                pre = plsc.cumsum(mi) - mi
                rp = (cnt + pre) & RMASK
                row = dl * R + t
                plsc.store_scatter(sel_src, [rp >> 7, rp & (B - 1)], s,
                                   mask=m)
                plsc.store_scatter(sel_row, [rp >> 7, rp & (B - 1)], row,
                                   mask=m)
                return cnt + jnp.sum(mi)
            cnt = lax.fori_loop(0, EBLK // 16, scan_body, cnt)
            kdone = flush(kdone, cnt >> 7)
            return (cnt, kdone)
        cnt, kdone = lax.fori_loop(0, NEB, block_body,
                                   (jnp.int32(0), jnp.int32(0)))

        nb = (cnt + (B - 1)) // B
        pad = nb * B - cnt
        for j in range(B // 16):
            p = (cnt + j * 16 + iota16) & RMASK
            mpad = (j * 16 + iota16) < pad
            plsc.store_scatter(sel_src, [p >> 7, p & (B - 1)],
                               sid * 16 + iota16, mask=mpad)
            plsc.store_scatter(sel_row, [p >> 7, p & (B - 1)],
                               ROWS + iota16, mask=mpad)
        flush(kdone, nb)
        plsc.subcore_barrier()

        hb = c * ROWS + db
        pltpu.sync_copy(acc_sh.at[pl.ds(db, RPT)],
                        s_hbm.at[pl.ds(hb, RPT)])
        return carry
    lax.fori_loop(0, K_PER_SC, chunk_body, 0)


def _make_sc_kernel():
    mesh = plsc.VectorSubcoreMesh(core_axis_name="c", subcore_axis_name="s",
                                  num_cores=NC, num_subcores=NS)
    return pl.kernel(
        _sc_body,
        out_type=(jax.ShapeDtypeStruct((SROWS, D), jnp.float32),),
        mesh=mesh,
        compiler_params=pltpu.CompilerParams(needs_layout_passes=False),
        scratch_types=(
            pltpu.VMEM_SHARED((ROWSP, D), jnp.float32),
            pltpu.VMEM((EBLK,), jnp.int32),
            pltpu.VMEM((EBLK,), jnp.int32),
            pltpu.VMEM((EBLK,), jnp.int32),
            pltpu.VMEM((RING, B), jnp.int32),
            pltpu.VMEM((RING, B), jnp.int32),
            pltpu.VMEM((B, D), jnp.float32),
        ),
    )


_sc_layer = _make_sc_kernel()

BN = 1000


def _tc_body(leaky, x_ref, s_ref, c_ref, root_ref, w_ref, b_ref, g_ref,
             bt_ref, o_ref):
    x = x_ref[...]
    acc = jnp.dot(x, root_ref[...], preferred_element_type=jnp.float32)
    acc = acc + b_ref[...]
    rec = jnp.concatenate(
        [jnp.broadcast_to(jnp.maximum(c_ref[:, r * D:r * D + 1], 1.0),
                          (BN, D)) for r in range(R)], axis=1)
    mean = s_ref[...] / rec
    acc = acc + jnp.dot(mean, w_ref[...], preferred_element_type=jnp.float32)
    if leaky:
        acc = jnp.where(acc > 0, acc, 0.2 * acc)
    mu = jnp.mean(acc, axis=1, keepdims=True)
    var = jnp.mean((acc - mu) ** 2, axis=1, keepdims=True)
    o_ref[...] = (acc - mu) / jnp.sqrt(var + EPS) * g_ref[...] + bt_ref[...]


def _make_tc_kernel(leaky):
    return pl.pallas_call(
        functools.partial(_tc_body, leaky),
        grid=(N // BN,),
        in_specs=[
            pl.BlockSpec((BN, D), lambda i: (i, 0)),
            pl.BlockSpec((BN, R * D), lambda i: (i, 0)),
            pl.BlockSpec((BN, R * D), lambda i: (i, 0)),
            pl.BlockSpec((D, D), lambda i: (0, 0)),
            pl.BlockSpec((R * D, D), lambda i: (0, 0)),
            pl.BlockSpec((1, D), lambda i: (0, 0)),
            pl.BlockSpec((1, D), lambda i: (0, 0)),
            pl.BlockSpec((1, D), lambda i: (0, 0)),
        ],
        out_specs=pl.BlockSpec((BN, D), lambda i: (i, 0)),
        out_shape=jax.ShapeDtypeStruct((N, D), jnp.float32),
    )


_tc_layer1 = _make_tc_kernel(True)
_tc_layer2 = _make_tc_kernel(False)


def kernel(x, edge_index, edge_type, W1, root1, b1, g1, beta1, W2, root2,
           b2, g2, beta2):
    src = edge_index[0]
    dst = edge_index[1]
    ones_x = jnp.ones((N, D), jnp.float32)
    (cnt,) = _sc_layer(ones_x, src, dst, edge_type)
    cs = cnt[:N * R].reshape(N, R * D)
    (s1,) = _sc_layer(x, src, dst, edge_type)
    h = _tc_layer1(x, s1[:N * R].reshape(N, R * D), cs, root1,
                   W1.reshape(R * D, D), b1.reshape(1, D),
                   g1.reshape(1, D), beta1.reshape(1, D))
    (s2,) = _sc_layer(h, src, dst, edge_type)
    out = _tc_layer2(h, s2[:N * R].reshape(N, R * D), cs, root2,
                     W2.reshape(R * D, D), b2.reshape(1, D),
                     g2.reshape(1, D), beta2.reshape(1, D))
    return out
